# baseline (speedup 1.0000x reference)
import functools

import jax
import jax.numpy as jnp
from jax import lax
from jax.experimental import pallas as pl
from jax.experimental.pallas import tpu as pltpu
from jax.experimental.pallas import tpu_sc as plsc

N_NODES = 50000
D = 512
B = 512
F1 = 25
F2 = 10
H1 = 512
OUT = 256

NW = 32
ROWS_W = (B * F1) // NW
CH = 8
GCH = CH * F2
NCH = ROWS_W // CH
NCH1 = ROWS_W // GCH
G0_W = B // NW
LG = D // 16


def _sc_gather_g1(x, hop1c, bn):
    mesh = plsc.VectorSubcoreMesh(core_axis_name="c", subcore_axis_name="s")

    @functools.partial(
        pl.kernel,
        mesh=mesh,
        out_type=[
            jax.ShapeDtypeStruct((B * F1, D), jnp.float32),
            jax.ShapeDtypeStruct((B, D), jnp.float32),
        ],
        scratch_types=[
            pltpu.VMEM((NCH1, GCH), jnp.int32),
            pltpu.VMEM((G0_W,), jnp.int32),
            pltpu.VMEM((GCH, D), jnp.float32),
            pltpu.VMEM((GCH, D), jnp.float32),
            pltpu.VMEM((G0_W, D), jnp.float32),
            pltpu.SemaphoreType.DMA,
            pltpu.SemaphoreType.DMA,
        ],
    )
    def k(x_hbm, h1_hbm, bn_hbm, g1_hbm, g0_hbm,
          idx1_v, idx0_v, gbufA, gbufB, g0buf, semA, semB):
        c_id = lax.axis_index("c")
        s_id = lax.axis_index("s")
        wid = s_id * 2 + c_id
        base = wid * ROWS_W
        pltpu.sync_copy(h1_hbm.at[wid], idx1_v)
        pltpu.sync_copy(bn_hbm.at[pl.ds(wid * G0_W, G0_W)], idx0_v)

        pltpu.async_copy(x_hbm.at[idx1_v.at[0]], gbufA, semA)
        for c in range(NCH1):
            buf, sem = (gbufA, semA) if c % 2 == 0 else (gbufB, semB)
            nbuf, nsem = (gbufB, semB) if c % 2 == 0 else (gbufA, semA)
            pltpu.make_async_copy(x_hbm.at[idx1_v.at[0]], buf, sem).wait()
            if c + 1 < NCH1:
                pltpu.async_copy(x_hbm.at[idx1_v.at[c + 1]], nbuf, nsem)
            pltpu.sync_copy(buf, g1_hbm.at[pl.ds(base + c * GCH, GCH)])

        pltpu.async_copy(x_hbm.at[idx0_v], g0buf, semA).wait()
        pltpu.sync_copy(g0buf, g0_hbm.at[pl.ds(wid * G0_W, G0_W)])

    return k(x, hop1c, bn)


def _sc_gather_s2(x, hop2c):
    mesh = plsc.VectorSubcoreMesh(core_axis_name="c", subcore_axis_name="s")

    @functools.partial(
        pl.kernel,
        mesh=mesh,
        out_type=jax.ShapeDtypeStruct((B * F1, D), jnp.float32),
        scratch_types=[
            pltpu.VMEM((NCH, GCH), jnp.int32),
            pltpu.VMEM((GCH, D), jnp.float32),
            pltpu.VMEM((GCH, D), jnp.float32),
            pltpu.VMEM((CH, D), jnp.float32),
            pltpu.VMEM((CH, D), jnp.float32),
            pltpu.SemaphoreType.DMA,
            pltpu.SemaphoreType.DMA,
            pltpu.SemaphoreType.DMA,
            pltpu.SemaphoreType.DMA,
        ],
    )
    def k(x_hbm, h2_hbm, s2_hbm, idx2_v, gbufA, gbufB, obufA, obufB,
          semA, semB, semOA, semOB):
        c_id = lax.axis_index("c")
        s_id = lax.axis_index("s")
        wid = s_id * 2 + c_id
        base = wid * ROWS_W
        pltpu.sync_copy(h2_hbm.at[wid], idx2_v)

        def start2(c, buf, sem):
            pltpu.async_copy(x_hbm.at[idx2_v.at[c]], buf, sem)

        def wait_g(buf, sem):
            pltpu.make_async_copy(x_hbm.at[idx2_v.at[0]], buf, sem).wait()

        def process(gbuf, obuf, semO, c):
            @pl.when(c >= 2)
            def _():
                pltpu.make_async_copy(obuf, s2_hbm.at[pl.ds(base, CH)],
                                      semO).wait()

            def lg_body(l, carry):
                sl = pl.ds(l * 16, 16)
                accs = [gbuf[o * F2, sl] for o in range(CH)]
                for j in range(1, F2):
                    for o in range(CH):
                        accs[o] = accs[o] + gbuf[o * F2 + j, sl]
                for o in range(CH):
                    obuf[o, sl] = accs[o] * (1.0 / F2)
                return carry

            lax.fori_loop(0, LG, lg_body, 0)
            pltpu.async_copy(obuf, s2_hbm.at[pl.ds(base + c * CH, CH)], semO)

        start2(0, gbufA, semA)

        def pair(p, carry):
            c0 = 2 * p
            wait_g(gbufA, semA)
            start2(c0 + 1, gbufB, semB)
            process(gbufA, obufA, semOA, c0)
            wait_g(gbufB, semB)

            @pl.when(c0 + 2 < NCH)
            def _():
                start2(c0 + 2, gbufA, semA)

            process(gbufB, obufB, semOB, c0 + 1)
            return carry

        lax.fori_loop(0, NCH // 2, pair, 0)
        pltpu.make_async_copy(obufA, s2_hbm.at[pl.ds(base, CH)], semOA).wait()
        pltpu.make_async_copy(obufB, s2_hbm.at[pl.ds(base, CH)], semOB).wait()

    return k(x, hop2c)


BLK = (B * F1) // 32
NB = B // 32


def _sel():
    grp = lax.broadcasted_iota(jnp.int32, (NB, BLK), 1) // F1
    row = lax.broadcasted_iota(jnp.int32, (NB, BLK), 0)
    return (grp == row).astype(jnp.float32)


def _tc_a_body(g1_ref, w1a_ref, b1_ref, a1_ref, g1s_ref):
    i = pl.program_id(0)
    g1b = g1_ref[...]
    a = jnp.dot(g1b.astype(jnp.bfloat16), w1a_ref[...],
                preferred_element_type=jnp.float32)
    a1_ref[...] = (a + b1_ref[...]).astype(jnp.bfloat16)
    g1s_ref[pl.ds(i * NB, NB), :] = jnp.dot(
        _sel(), g1b, preferred_element_type=jnp.float32)


def _tc_a(g1, w1aT, b1r):
    return pl.pallas_call(
        _tc_a_body,
        grid=(32,),
        in_specs=[
            pl.BlockSpec((BLK, D), lambda i: (i, 0)),
            pl.BlockSpec((D, H1), lambda i: (0, 0)),
            pl.BlockSpec((1, H1), lambda i: (0, 0)),
        ],
        out_specs=[
            pl.BlockSpec((BLK, H1), lambda i: (i, 0)),
            pl.BlockSpec((B, D), lambda i: (0, 0)),
        ],
        out_shape=[
            jax.ShapeDtypeStruct((B * F1, H1), jnp.bfloat16),
            jax.ShapeDtypeStruct((B, D), jnp.float32),
        ],
    )(g1, w1aT, b1r)


def _tc_b_body(a1_ref, s2_ref, g1s_ref, g0_ref, w1a_ref, w1b_ref, b1_ref,
               w2a_ref, w2b_ref, b2_ref, out_ref, h1s_ref):
    i = pl.program_id(0)
    a = a1_ref[...].astype(jnp.float32) + jnp.dot(
        s2_ref[...].astype(jnp.bfloat16), w1b_ref[...],
        preferred_element_type=jnp.float32)
    h = jnp.maximum(a, 0.0)
    n2 = jnp.sum(h * h, axis=1, keepdims=True)
    h = h * jax.lax.rsqrt(jnp.maximum(n2, 1e-24))
    h1s_ref[pl.ds(i * NB, NB), :] = jnp.dot(
        _sel(), h, preferred_element_type=jnp.float32)

    @pl.when(i == 31)
    def _():
        m1 = g1s_ref[...] * (1.0 / F1)
        h0 = jnp.dot(g0_ref[...].astype(jnp.bfloat16), w1a_ref[...],
                     preferred_element_type=jnp.float32)
        h0 = h0 + jnp.dot(m1.astype(jnp.bfloat16), w1b_ref[...],
                          preferred_element_type=jnp.float32)
        h0 = jnp.maximum(h0 + b1_ref[...], 0.0)
        n0 = jnp.sum(h0 * h0, axis=1, keepdims=True)
        h0 = h0 * jax.lax.rsqrt(jnp.maximum(n0, 1e-24))
        h1m = h1s_ref[...] * (1.0 / F1)
        o = jnp.dot(h0, w2a_ref[...], preferred_element_type=jnp.float32)
        o = o + jnp.dot(h1m, w2b_ref[...], preferred_element_type=jnp.float32)
        o = jnp.maximum(o + b2_ref[...], 0.0)
        no = jnp.sum(o * o, axis=1, keepdims=True)
        out_ref[...] = o * jax.lax.rsqrt(jnp.maximum(no, 1e-24))


def _tc_b(a1, s2, g1s, g0, w1aT, w1bT, b1r, w2aT, w2bT, b2r):
    return pl.pallas_call(
        _tc_b_body,
        grid=(32,),
        in_specs=[
            pl.BlockSpec((BLK, H1), lambda i: (i, 0)),
            pl.BlockSpec((BLK, D), lambda i: (i, 0)),
            pl.BlockSpec((B, D), lambda i: (0, 0)),
            pl.BlockSpec((B, D), lambda i: (0, 0)),
            pl.BlockSpec((D, H1), lambda i: (0, 0)),
            pl.BlockSpec((D, H1), lambda i: (0, 0)),
            pl.BlockSpec((1, H1), lambda i: (0, 0)),
            pl.BlockSpec((H1, OUT), lambda i: (0, 0)),
            pl.BlockSpec((H1, OUT), lambda i: (0, 0)),
            pl.BlockSpec((1, OUT), lambda i: (0, 0)),
        ],
        out_specs=pl.BlockSpec((B, OUT), lambda i: (0, 0)),
        out_shape=jax.ShapeDtypeStruct((B, OUT), jnp.float32),
        scratch_shapes=[
            pltpu.VMEM((B, H1), jnp.float32),
        ],
    )(a1, s2, g1s, g0, w1aT, w1bT, b1r, w2aT, w2bT, b2r)


def kernel(x, batch_nodes, hop1, hop2, W1, b1, W2, b2):
    hop2c = hop2.astype(jnp.int32).reshape(NW, NCH, GCH)
    hop1c = hop1.astype(jnp.int32).reshape(NW, NCH1, GCH)
    bn = batch_nodes.astype(jnp.int32)
    w1aT = W1[:, :D].T.astype(jnp.bfloat16)
    w1bT = W1[:, D:].T.astype(jnp.bfloat16)
    w2aT = W2[:, :H1].T
    w2bT = W2[:, H1:].T
    b1r = b1.reshape(1, H1)
    b2r = b2.reshape(1, OUT)
    g1, g0 = _sc_gather_g1(x, hop1c, bn)
    s2 = _sc_gather_s2(x, hop2c)
    a1, g1s = _tc_a(g1, w1aT, b1r)
    return _tc_b(a1, s2, g1s, g0, w1aT, w1bT, b1r, w2aT, w2bT, b2r)


# speedup vs baseline: 1.0044x; 1.0044x over previous
import functools

import jax
import jax.numpy as jnp
from jax import lax
from jax.experimental import pallas as pl
from jax.experimental.pallas import tpu as pltpu
from jax.experimental.pallas import tpu_sc as plsc

N_NODES = 50000
D = 512
B = 512
F1 = 25
F2 = 10
H1 = 512
OUT = 256

NW = 32
ROWS_W = (B * F1) // NW
CH = 8
GCH = CH * F2
NCH = ROWS_W // CH
NCH1 = ROWS_W // GCH
G0_W = B // NW
LG = D // 16


def _sc_gather_g1(x, hop1c, bn):
    mesh = plsc.VectorSubcoreMesh(core_axis_name="c", subcore_axis_name="s")

    @functools.partial(
        pl.kernel,
        mesh=mesh,
        out_type=[
            jax.ShapeDtypeStruct((B * F1, D), jnp.float32),
            jax.ShapeDtypeStruct((B, D), jnp.float32),
        ],
        scratch_types=[
            pltpu.VMEM((NCH1, GCH), jnp.int32),
            pltpu.VMEM((G0_W,), jnp.int32),
            pltpu.VMEM((GCH, D), jnp.float32),
            pltpu.VMEM((GCH, D), jnp.float32),
            pltpu.VMEM((G0_W, D), jnp.float32),
            pltpu.SemaphoreType.DMA,
            pltpu.SemaphoreType.DMA,
            pltpu.SemaphoreType.DMA,
            pltpu.SemaphoreType.DMA,
        ],
    )
    def k(x_hbm, h1_hbm, bn_hbm, g1_hbm, g0_hbm,
          idx1_v, idx0_v, gbufA, gbufB, g0buf, semA, semB, semWA, semWB):
        c_id = lax.axis_index("c")
        s_id = lax.axis_index("s")
        wid = s_id * 2 + c_id
        base = wid * ROWS_W
        pltpu.sync_copy(h1_hbm.at[wid], idx1_v)
        pltpu.sync_copy(bn_hbm.at[pl.ds(wid * G0_W, G0_W)], idx0_v)

        pltpu.async_copy(x_hbm.at[idx1_v.at[0]], gbufA, semA)
        for c in range(NCH1):
            buf, sem, semW = ((gbufA, semA, semWA) if c % 2 == 0
                              else (gbufB, semB, semWB))
            nbuf, nsem, nsemW = ((gbufB, semB, semWB) if c % 2 == 0
                                 else (gbufA, semA, semWA))
            pltpu.make_async_copy(x_hbm.at[idx1_v.at[0]], buf, sem).wait()
            if c + 1 < NCH1:
                if c >= 1:
                    pltpu.make_async_copy(
                        nbuf, g1_hbm.at[pl.ds(base, GCH)], nsemW).wait()
                pltpu.async_copy(x_hbm.at[idx1_v.at[c + 1]], nbuf, nsem)
            pltpu.async_copy(buf, g1_hbm.at[pl.ds(base + c * GCH, GCH)], semW)
        pltpu.make_async_copy(gbufB, g1_hbm.at[pl.ds(base, GCH)], semWB).wait()
        pltpu.make_async_copy(gbufA, g1_hbm.at[pl.ds(base, GCH)], semWA).wait()

        pltpu.async_copy(x_hbm.at[idx0_v], g0buf, semA).wait()
        pltpu.sync_copy(g0buf, g0_hbm.at[pl.ds(wid * G0_W, G0_W)])

    return k(x, hop1c, bn)


def _sc_gather_s2(x, hop2c):
    mesh = plsc.VectorSubcoreMesh(core_axis_name="c", subcore_axis_name="s")

    @functools.partial(
        pl.kernel,
        mesh=mesh,
        out_type=jax.ShapeDtypeStruct((B * F1, D), jnp.float32),
        scratch_types=[
            pltpu.VMEM((NCH, GCH), jnp.int32),
            pltpu.VMEM((GCH, D), jnp.float32),
            pltpu.VMEM((GCH, D), jnp.float32),
            pltpu.VMEM((CH, D), jnp.float32),
            pltpu.VMEM((CH, D), jnp.float32),
            pltpu.SemaphoreType.DMA,
            pltpu.SemaphoreType.DMA,
            pltpu.SemaphoreType.DMA,
            pltpu.SemaphoreType.DMA,
        ],
    )
    def k(x_hbm, h2_hbm, s2_hbm, idx2_v, gbufA, gbufB, obufA, obufB,
          semA, semB, semOA, semOB):
        c_id = lax.axis_index("c")
        s_id = lax.axis_index("s")
        wid = s_id * 2 + c_id
        base = wid * ROWS_W
        pltpu.sync_copy(h2_hbm.at[wid], idx2_v)

        def start2(c, buf, sem):
            pltpu.async_copy(x_hbm.at[idx2_v.at[c]], buf, sem)

        def wait_g(buf, sem):
            pltpu.make_async_copy(x_hbm.at[idx2_v.at[0]], buf, sem).wait()

        def process(gbuf, obuf, semO, c):
            @pl.when(c >= 2)
            def _():
                pltpu.make_async_copy(obuf, s2_hbm.at[pl.ds(base, CH)],
                                      semO).wait()

            def lg_body(l, carry):
                sl = pl.ds(l * 16, 16)
                accs = [gbuf[o * F2, sl] for o in range(CH)]
                for j in range(1, F2):
                    for o in range(CH):
                        accs[o] = accs[o] + gbuf[o * F2 + j, sl]
                for o in range(CH):
                    obuf[o, sl] = accs[o] * (1.0 / F2)
                return carry

            lax.fori_loop(0, LG, lg_body, 0)
            pltpu.async_copy(obuf, s2_hbm.at[pl.ds(base + c * CH, CH)], semO)

        start2(0, gbufA, semA)

        def pair(p, carry):
            c0 = 2 * p
            wait_g(gbufA, semA)
            start2(c0 + 1, gbufB, semB)
            process(gbufA, obufA, semOA, c0)
            wait_g(gbufB, semB)

            @pl.when(c0 + 2 < NCH)
            def _():
                start2(c0 + 2, gbufA, semA)

            process(gbufB, obufB, semOB, c0 + 1)
            return carry

        lax.fori_loop(0, NCH // 2, pair, 0)
        pltpu.make_async_copy(obufA, s2_hbm.at[pl.ds(base, CH)], semOA).wait()
        pltpu.make_async_copy(obufB, s2_hbm.at[pl.ds(base, CH)], semOB).wait()

    return k(x, hop2c)


BLK = (B * F1) // 32
NB = B // 32


def _sel():
    grp = lax.broadcasted_iota(jnp.int32, (NB, BLK), 1) // F1
    row = lax.broadcasted_iota(jnp.int32, (NB, BLK), 0)
    return (grp == row).astype(jnp.float32)


def _tc_a_body(g1_ref, w1a_ref, b1_ref, a1_ref, g1s_ref):
    i = pl.program_id(0)
    g1b = g1_ref[...]
    a = jnp.dot(g1b.astype(jnp.bfloat16), w1a_ref[...],
                preferred_element_type=jnp.float32)
    a1_ref[...] = (a + b1_ref[...]).astype(jnp.bfloat16)
    g1s_ref[pl.ds(i * NB, NB), :] = jnp.dot(
        _sel(), g1b, preferred_element_type=jnp.float32)


def _tc_a(g1, w1aT, b1r):
    return pl.pallas_call(
        _tc_a_body,
        grid=(32,),
        in_specs=[
            pl.BlockSpec((BLK, D), lambda i: (i, 0)),
            pl.BlockSpec((D, H1), lambda i: (0, 0)),
            pl.BlockSpec((1, H1), lambda i: (0, 0)),
        ],
        out_specs=[
            pl.BlockSpec((BLK, H1), lambda i: (i, 0)),
            pl.BlockSpec((B, D), lambda i: (0, 0)),
        ],
        out_shape=[
            jax.ShapeDtypeStruct((B * F1, H1), jnp.bfloat16),
            jax.ShapeDtypeStruct((B, D), jnp.float32),
        ],
    )(g1, w1aT, b1r)


def _tc_b_body(a1_ref, s2_ref, g1s_ref, g0_ref, w1a_ref, w1b_ref, b1_ref,
               w2a_ref, w2b_ref, b2_ref, out_ref, h1s_ref):
    i = pl.program_id(0)
    a = a1_ref[...].astype(jnp.float32) + jnp.dot(
        s2_ref[...].astype(jnp.bfloat16), w1b_ref[...],
        preferred_element_type=jnp.float32)
    h = jnp.maximum(a, 0.0)
    n2 = jnp.sum(h * h, axis=1, keepdims=True)
    h = h * jax.lax.rsqrt(jnp.maximum(n2, 1e-24))
    h1s_ref[pl.ds(i * NB, NB), :] = jnp.dot(
        _sel(), h, preferred_element_type=jnp.float32)

    @pl.when(i == 31)
    def _():
        m1 = g1s_ref[...] * (1.0 / F1)
        h0 = jnp.dot(g0_ref[...].astype(jnp.bfloat16), w1a_ref[...],
                     preferred_element_type=jnp.float32)
        h0 = h0 + jnp.dot(m1.astype(jnp.bfloat16), w1b_ref[...],
                          preferred_element_type=jnp.float32)
        h0 = jnp.maximum(h0 + b1_ref[...], 0.0)
        n0 = jnp.sum(h0 * h0, axis=1, keepdims=True)
        h0 = h0 * jax.lax.rsqrt(jnp.maximum(n0, 1e-24))
        h1m = h1s_ref[...] * (1.0 / F1)
        o = jnp.dot(h0, w2a_ref[...], preferred_element_type=jnp.float32)
        o = o + jnp.dot(h1m, w2b_ref[...], preferred_element_type=jnp.float32)
        o = jnp.maximum(o + b2_ref[...], 0.0)
        no = jnp.sum(o * o, axis=1, keepdims=True)
        out_ref[...] = o * jax.lax.rsqrt(jnp.maximum(no, 1e-24))


def _tc_b(a1, s2, g1s, g0, w1aT, w1bT, b1r, w2aT, w2bT, b2r):
    return pl.pallas_call(
        _tc_b_body,
        grid=(32,),
        in_specs=[
            pl.BlockSpec((BLK, H1), lambda i: (i, 0)),
            pl.BlockSpec((BLK, D), lambda i: (i, 0)),
            pl.BlockSpec((B, D), lambda i: (0, 0)),
            pl.BlockSpec((B, D), lambda i: (0, 0)),
            pl.BlockSpec((D, H1), lambda i: (0, 0)),
            pl.BlockSpec((D, H1), lambda i: (0, 0)),
            pl.BlockSpec((1, H1), lambda i: (0, 0)),
            pl.BlockSpec((H1, OUT), lambda i: (0, 0)),
            pl.BlockSpec((H1, OUT), lambda i: (0, 0)),
            pl.BlockSpec((1, OUT), lambda i: (0, 0)),
        ],
        out_specs=pl.BlockSpec((B, OUT), lambda i: (0, 0)),
        out_shape=jax.ShapeDtypeStruct((B, OUT), jnp.float32),
        scratch_shapes=[
            pltpu.VMEM((B, H1), jnp.float32),
        ],
    )(a1, s2, g1s, g0, w1aT, w1bT, b1r, w2aT, w2bT, b2r)


def kernel(x, batch_nodes, hop1, hop2, W1, b1, W2, b2):
    hop2c = hop2.astype(jnp.int32).reshape(NW, NCH, GCH)
    hop1c = hop1.astype(jnp.int32).reshape(NW, NCH1, GCH)
    bn = batch_nodes.astype(jnp.int32)
    w1aT = W1[:, :D].T.astype(jnp.bfloat16)
    w1bT = W1[:, D:].T.astype(jnp.bfloat16)
    w2aT = W2[:, :H1].T
    w2bT = W2[:, H1:].T
    b1r = b1.reshape(1, H1)
    b2r = b2.reshape(1, OUT)
    g1, g0 = _sc_gather_g1(x, hop1c, bn)
    s2 = _sc_gather_s2(x, hop2c)
    a1, g1s = _tc_a(g1, w1aT, b1r)
    return _tc_b(a1, s2, g1s, g0, w1aT, w1bT, b1r, w2aT, w2bT, b2r)
